# Initial kernel scaffold; baseline (speedup 1.0000x reference)
import jax
import jax.numpy as jnp
from jax.experimental import pallas as pl


def kernel(x, edge_index, W_self0, W_neigh0, b0, W_self1, W_neigh1, b1):
    raise NotImplementedError("write your pallas kernel here")


# speedup vs baseline: 6.0860x; 6.0860x over previous
import functools

import jax
import jax.numpy as jnp
from jax import lax
from jax.experimental import pallas as pl
from jax.experimental.pallas import tpu as pltpu
from jax.experimental.pallas import tpu_sc as plsc

N = 10000
E = 320000
D = 128
NC = 2
NS = 16
NW = NC * NS
K = 128
NCHUNKS = E // K
BASE_CH = NCHUNKS // NW
EXTRA = NCHUNKS - BASE_CH * NW
RPT = 624
RTAIL = N - NS * RPT
DW = 16


def _sc_body(want_deg, *refs):
    if want_deg:
        (x_hbm, src_hbm, dst_hbm, z2d, z1d, ones_hbm,
         agg_out, deg_out, src_v, dst_v, rows_v, ones_v, agg_sh, deg_sh) = refs
    else:
        (x_hbm, src_hbm, dst_hbm, z2d,
         agg_out, src_v, dst_v, rows_v, agg_sh) = refs

    cid = lax.axis_index("c")
    sid = lax.axis_index("s")
    wid = cid * NS + sid

    pltpu.sync_copy(z2d.at[pl.ds(sid * RPT, RPT)], agg_sh.at[pl.ds(sid * RPT, RPT)])
    if want_deg:
        pltpu.sync_copy(z1d.at[pl.ds(sid * RPT, RPT)], deg_sh.at[pl.ds(sid * RPT, RPT)])
        pltpu.sync_copy(ones_hbm, ones_v)

    @pl.when(sid == NS - 1)
    def _():
        t0 = pl.ds(NS * RPT, RTAIL)
        pltpu.sync_copy(z2d.at[t0], agg_sh.at[t0])
        if want_deg:
            pltpu.sync_copy(z1d.at[t0], deg_sh.at[t0])

    plsc.subcore_barrier()

    nch = BASE_CH + jnp.where(wid < EXTRA, 1, 0)

    def chunk(j, carry):
        b = (j * NW + wid) * K
        pltpu.sync_copy(src_hbm.at[pl.ds(b, K)], src_v)
        pltpu.sync_copy(dst_hbm.at[pl.ds(b, K)], dst_v)
        pltpu.sync_copy(x_hbm.at[src_v], rows_v)
        pltpu.sync_copy(rows_v, agg_sh.at[dst_v], add=True)
        if want_deg:
            pltpu.sync_copy(ones_v, deg_sh.at[dst_v], add=True)
        return carry

    lax.fori_loop(0, nch, chunk, 0)
    plsc.subcore_barrier()

    r0 = pl.ds(sid * RPT, RPT)
    pltpu.sync_copy(agg_sh.at[r0], agg_out.at[pl.ds(cid * N + sid * RPT, RPT)])
    if want_deg:
        pltpu.sync_copy(deg_sh.at[r0], deg_out.at[pl.ds(cid * N + sid * RPT, RPT)])

    @pl.when(sid == NS - 1)
    def _():
        t0 = pl.ds(NS * RPT, RTAIL)
        to = pl.ds(cid * N + NS * RPT, RTAIL)
        pltpu.sync_copy(agg_sh.at[t0], agg_out.at[to])
        if want_deg:
            pltpu.sync_copy(deg_sh.at[t0], deg_out.at[to])


def _make_sc(want_deg):
    mesh = plsc.VectorSubcoreMesh(core_axis_name="c", subcore_axis_name="s")
    out_type = [jax.ShapeDtypeStruct((NC * N, D), jnp.float32)]
    scratch = [
        pltpu.VMEM((K,), jnp.int32),
        pltpu.VMEM((K,), jnp.int32),
        pltpu.VMEM((K, D), jnp.float32),
    ]
    shared = [pltpu.VMEM_SHARED((N, D), jnp.float32)]
    if want_deg:
        out_type.append(jax.ShapeDtypeStruct((NC * N, DW), jnp.float32))
        scratch.append(pltpu.VMEM((K, DW), jnp.float32))
        shared.append(pltpu.VMEM_SHARED((N, DW), jnp.float32))
    return pl.kernel(
        functools.partial(_sc_body, want_deg),
        out_type=tuple(out_type) if want_deg else out_type[0],
        mesh=mesh,
        scratch_types=scratch + shared,
        compiler_params=pltpu.CompilerParams(use_tc_tiling_on_sc=False),
    )


_sc_agg_deg = _make_sc(True)
_sc_agg = _make_sc(False)


def _tc_layer(x, aggp, degp, w_self, w_neigh, b, relu):
    nb = 10
    br = N // nb

    def body(x_ref, aggp_ref, degp_ref, ws_ref, wn_ref, b_ref, o_ref):
        agg = aggp_ref[0] + aggp_ref[1]
        deg = degp_ref[0, :, 0:1] + degp_ref[1, :, 0:1]
        h = agg / jnp.maximum(deg, 1.0)
        o = (jnp.dot(x_ref[...], ws_ref[...], preferred_element_type=jnp.float32)
             + jnp.dot(h, wn_ref[...], preferred_element_type=jnp.float32)
             + b_ref[...])
        o_ref[...] = jnp.maximum(o, 0.0) if relu else o

    return pl.pallas_call(
        body,
        grid=(nb,),
        in_specs=[
            pl.BlockSpec((br, D), lambda i: (i, 0)),
            pl.BlockSpec((2, br, D), lambda i: (0, i, 0)),
            pl.BlockSpec((2, br, DW), lambda i: (0, i, 0)),
            pl.BlockSpec((D, D), lambda i: (0, 0)),
            pl.BlockSpec((D, D), lambda i: (0, 0)),
            pl.BlockSpec((1, D), lambda i: (0, 0)),
        ],
        out_specs=pl.BlockSpec((br, D), lambda i: (i, 0)),
        out_shape=jax.ShapeDtypeStruct((N, D), jnp.float32),
    )(x, aggp.reshape(2, N, D), degp.reshape(2, N, DW), w_self, w_neigh,
      b.reshape(1, D))


def kernel(x, edge_index, W_self0, W_neigh0, b0, W_self1, W_neigh1, b1):
    src = edge_index[0].astype(jnp.int32)
    dst = edge_index[1].astype(jnp.int32)
    z2d = jnp.zeros((N, D), jnp.float32)
    z1d = jnp.zeros((N, DW), jnp.float32)
    ones = jnp.ones((K, DW), jnp.float32)

    aggp0, degp = _sc_agg_deg(x, src, dst, z2d, z1d, ones)
    h = _tc_layer(x, aggp0, degp, W_self0, W_neigh0, b0, relu=True)
    aggp1 = _sc_agg(h, src, dst, z2d)
    return _tc_layer(h, aggp1, degp, W_self1, W_neigh1, b1, relu=False)
